# Initial kernel scaffold; baseline (speedup 1.0000x reference)
import jax
import jax.numpy as jnp
from jax.experimental import pallas as pl


def kernel(query, key, proj_dir_score):
    raise NotImplementedError("write your pallas kernel here")


# speedup vs baseline: 2.8683x; 2.8683x over previous
import math
import jax
import jax.numpy as jnp
from jax.experimental import pallas as pl
from jax.experimental.pallas import tpu as pltpu


def _qjl_score_kernel(q_ref, k_ref, p_ref, out_ref, *, scale):
    q = q_ref[0, 0]
    k = k_ref[0, 0]
    p = p_ref[...]

    prec = jax.lax.Precision.DEFAULT
    sq = jax.lax.dot_general(q, p, (((1,), (0,)), ((), ())),
                             preferred_element_type=jnp.float32,
                             precision=prec)
    sk = jax.lax.dot_general(k, p, (((1,), (0,)), ((), ())),
                             preferred_element_type=jnp.float32,
                             precision=prec)
    sgn = jnp.sign(sk)
    scores = jax.lax.dot_general(sgn, sq, (((1,), (1,)), ((), ())),
                                 preferred_element_type=jnp.float32,
                                 precision=prec)
    norm = jnp.sqrt(jnp.sum(k * k, axis=1, keepdims=True))
    out_ref[0, 0] = scores * (norm * scale)


def kernel(query, key, proj_dir_score):
    B, HQ, QL, D = query.shape
    _, HK, KV, _ = key.shape
    S = proj_dir_score.shape[1]
    n_rep = HQ // HK
    scale = math.sqrt(math.pi / 2.0) / float(S)

    q4 = query.reshape(B, HK, n_rep, D)

    out = pl.pallas_call(
        lambda qr, kr, pr, orf: _qjl_score_kernel(qr, kr, pr, orf, scale=scale),
        grid=(B, HK),
        in_specs=[
            pl.BlockSpec((1, 1, n_rep, D), lambda b, h: (b, h, 0, 0)),
            pl.BlockSpec((1, 1, KV, D), lambda b, h: (b, h, 0, 0)),
            pl.BlockSpec((D, S), lambda b, h: (0, 0)),
        ],
        out_specs=pl.BlockSpec((1, 1, KV, n_rep), lambda b, h: (b, h, 0, 0)),
        out_shape=jax.ShapeDtypeStruct((B, HK, KV, n_rep), jnp.float32),
        compiler_params=pltpu.CompilerParams(
            dimension_semantics=("parallel", "parallel"),
        ),
    )(q4, key, proj_dir_score)

    return out.transpose(0, 1, 3, 2).reshape(B, HQ, KV, 1)
